# baseline (speedup 1.0000x reference)
import functools

import jax
import jax.numpy as jnp
from jax import lax
from jax.experimental import pallas as pl
from jax.experimental.pallas import tpu as pltpu
from jax.experimental.pallas import tpu_sc as plsc

N_ROWS = 16384
N_COLS = 1000
L = 16
NC = 2
NS = 16
NW = NC * NS
ROWS_PER_W = N_ROWS // NW
CHUNK = 128
N_CHUNKS = ROWS_PER_W // CHUNK
N_VECS = ROWS_PER_W // L


def _ganloss_body(probs_hbm, tgt_hbm, rwd_hbm, out_hbm,
                  tgt_v, rwd_v, idx_v, val_v, acc_v, red_v, shared, sem):
    c = lax.axis_index("c")
    s = lax.axis_index("s")
    w = c * NS + s
    base = w * ROWS_PER_W

    pltpu.sync_copy(tgt_hbm.at[pl.ds(base, ROWS_PER_W)], tgt_v)
    pltpu.sync_copy(rwd_hbm.at[pl.ds(base, ROWS_PER_W)], rwd_v)

    lane = lax.iota(jnp.int32, L)
    for j in range(N_VECS):
        t = tgt_v[pl.ds(j * L, L)]
        row = (base + j * L) + lane
        idx_v[pl.ds(j * L, L)] = row * N_COLS + t

    copies = [
        pltpu.async_copy(
            probs_hbm.at[idx_v.at[pl.ds(k * CHUNK, CHUNK)]],
            val_v.at[pl.ds(k * CHUNK, CHUNK)],
            sem,
        )
        for k in range(N_CHUNKS)
    ]
    for cp in copies:
        cp.wait()

    acc = jnp.zeros((L,), jnp.float32)
    for j in range(N_VECS):
        v = val_v[pl.ds(j * L, L)]
        r = rwd_v[pl.ds(j * L, L)]
        acc = acc + (1.0 - v) * r
    acc_v[...] = acc * (1.0 / N_ROWS)

    pltpu.sync_copy(acc_v, shared.at[pl.ds(s * L, L)])
    plsc.subcore_barrier()

    @pl.when(s == 0)
    def _reduce():
        pltpu.sync_copy(shared, red_v)
        tot = jnp.zeros((L,), jnp.float32)
        for k in range(NS):
            tot = tot + red_v[pl.ds(k * L, L)]
        acc_v[...] = tot
        pltpu.sync_copy(acc_v, out_hbm.at[pl.ds(c * L, L)])


_ganloss_sc = functools.partial(
    pl.kernel,
    out_type=jax.ShapeDtypeStruct((NC * L,), jnp.float32),
    mesh=plsc.VectorSubcoreMesh(core_axis_name="c", subcore_axis_name="s"),
    scratch_types=[
        pltpu.VMEM((ROWS_PER_W,), jnp.int32),
        pltpu.VMEM((ROWS_PER_W,), jnp.float32),
        pltpu.VMEM((ROWS_PER_W,), jnp.int32),
        pltpu.VMEM((ROWS_PER_W,), jnp.float32),
        pltpu.VMEM((L,), jnp.float32),
        pltpu.VMEM((NS * L,), jnp.float32),
        pltpu.VMEM_SHARED((NS * L,), jnp.float32),
        pltpu.SemaphoreType.DMA,
    ],
)(_ganloss_body)


def kernel(probs, targets, reward):
    probs_flat = probs.reshape(-1)
    partials = _ganloss_sc(probs_flat, targets.astype(jnp.int32), reward)
    return jnp.sum(partials)


# speedup vs baseline: 1.3853x; 1.3853x over previous
import functools

import jax
import jax.numpy as jnp
from jax.experimental import pallas as pl
from jax.experimental.pallas import tpu as pltpu

N_ROWS = 16384
N_COLS = 1000
BLK = 512
GRID = N_ROWS // BLK


def _ganloss_tc_body(tgt_ref, rwd_ref, probs_ref, out_ref):
    g = pl.program_id(0)
    p = probs_ref[...]
    t = tgt_ref[...]
    r = rwd_ref[...]
    cols = jax.lax.broadcasted_iota(jnp.int32, (BLK, N_COLS), 1)
    sel = jnp.where(cols == t, p, 0.0).sum(axis=1, keepdims=True)
    part = jnp.sum((1.0 - sel) * r) * (1.0 / N_ROWS)

    @pl.when(g == 0)
    def _init():
        out_ref[0, 0] = 0.0

    out_ref[0, 0] += part


_ganloss_tc = pl.pallas_call(
    _ganloss_tc_body,
    grid=(GRID,),
    in_specs=[
        pl.BlockSpec((BLK, 1), lambda g: (g, 0)),
        pl.BlockSpec((BLK, 1), lambda g: (g, 0)),
        pl.BlockSpec((BLK, N_COLS), lambda g: (g, 0)),
    ],
    out_specs=pl.BlockSpec((1, 1), lambda g: (0, 0), memory_space=pltpu.SMEM),
    out_shape=jax.ShapeDtypeStruct((1, 1), jnp.float32),
    compiler_params=pltpu.CompilerParams(
        dimension_semantics=("arbitrary",),
    ),
)


def kernel(probs, targets, reward):
    t2 = targets.astype(jnp.int32).reshape(N_ROWS, 1)
    r2 = reward.reshape(N_ROWS, 1)
    out = _ganloss_tc(t2, r2, probs)
    return out[0, 0]
